# Initial kernel scaffold; baseline (speedup 1.0000x reference)
import jax
import jax.numpy as jnp
from jax.experimental import pallas as pl


def kernel(x, Wg1, bg1, Wg2, bg2, W1, b1, W2, b2, task_bh):
    raise NotImplementedError("write your pallas kernel here")


# speedup vs baseline: 2.7683x; 2.7683x over previous
import functools

import jax
import jax.numpy as jnp
from jax.experimental import pallas as pl
from jax.experimental.pallas import tpu as pltpu

N = 2048
D = 768
GH = D // 4
E = 8
H = 768

_INV_SQRT2 = 0.7071067811865476


def _gelu(v):
    return v * 0.5 * (1.0 + jax.lax.erf(v * _INV_SQRT2))


def _gate_body(x_ref, wg1_ref, bg1_ref, wg2_ref, bg2_ref, gates_ref):
    x = x_ref[...]
    g = jnp.dot(x, wg1_ref[...], preferred_element_type=jnp.float32) + bg1_ref[...]
    g = _gelu(g)
    logits = jnp.dot(g, wg2_ref[...], preferred_element_type=jnp.float32) + bg2_ref[...]
    m = jnp.max(logits, axis=-1, keepdims=True)
    p = jnp.exp(logits - m)
    probs = p / jnp.sum(p, axis=-1, keepdims=True)
    lane = jax.lax.broadcasted_iota(jnp.int32, (N, E), 1)
    v0 = jnp.max(probs, axis=-1, keepdims=True)
    i0 = jnp.min(jnp.where(probs == v0, lane, E), axis=-1, keepdims=True)
    probs1 = jnp.where(lane == i0, -1.0, probs)
    v1 = jnp.max(probs1, axis=-1, keepdims=True)
    i1 = jnp.min(jnp.where(probs1 == v1, lane, E), axis=-1, keepdims=True)
    gates = jnp.where(lane == i0, v0, 0.0) + jnp.where(lane == i1, v1, 0.0)
    gates_ref[...] = gates


TB = 1024
NTB = N // TB


def _ffn_body(x_ref, gates_ref, w1_ref, b1_ref, w2_ref, b2_ref, y_ref):
    e = pl.program_id(0)

    @pl.when(e == 0)
    def _init():
        y_ref[...] = jnp.zeros_like(y_ref)

    h = jnp.dot(x_ref[...], w1_ref[0], preferred_element_type=jnp.float32) + b1_ref[0, 0]
    h = _gelu(h)
    o = jnp.dot(h, w2_ref[0], preferred_element_type=jnp.float32) + b2_ref[0, 0]
    onehot = (jax.lax.broadcasted_iota(jnp.int32, (E, 1), 0) == e).astype(jnp.float32)
    col = jnp.dot(gates_ref[...], onehot, preferred_element_type=jnp.float32)
    y_ref[...] += col * o


@jax.jit
def kernel(x, Wg1, bg1, Wg2, bg2, W1, b1, W2, b2, task_bh):
    bsz, length, d = x.shape
    xf = x.reshape(N, D)

    gates = pl.pallas_call(
        _gate_body,
        out_shape=jax.ShapeDtypeStruct((N, E), jnp.float32),
    )(xf, Wg1, bg1.reshape(1, GH), Wg2, bg2.reshape(1, E))

    y = pl.pallas_call(
        _ffn_body,
        grid=(E, NTB),
        in_specs=[
            pl.BlockSpec((TB, D), lambda e, t: (t, 0)),
            pl.BlockSpec((TB, E), lambda e, t: (t, 0)),
            pl.BlockSpec((1, D, H), lambda e, t: (e, 0, 0)),
            pl.BlockSpec((1, 1, H), lambda e, t: (e, 0, 0)),
            pl.BlockSpec((1, H, D), lambda e, t: (e, 0, 0)),
            pl.BlockSpec((1, 1, D), lambda e, t: (e, 0, 0)),
        ],
        out_specs=pl.BlockSpec((TB, D), lambda e, t: (t, 0)),
        out_shape=jax.ShapeDtypeStruct((N, D), jnp.float32),
    )(xf, gates, W1, b1.reshape(E, 1, H), W2, b2.reshape(E, 1, D))

    return y.reshape(bsz, length, d)
